# baseline (speedup 1.0000x reference)
import jax
import jax.numpy as jnp
import numpy as np
from jax import lax
from jax.experimental import pallas as pl
from jax.experimental.pallas import tpu as pltpu
from jax.experimental.pallas import tpu_sc as plsc
from jax.sharding import Mesh, NamedSharding, PartitionSpec as P

try:
    from jax import shard_map as _shard_map
except ImportError:
    from jax.experimental.shard_map import shard_map as _shard_map

EMBEDDING_DIM = 128
WINDOW = 128
NBUF = 4
NUM_CORES = 2
NUM_SUBCORES = 16
NUM_WORKERS = NUM_CORES * NUM_SUBCORES


def _sc_gather(table, idx2d):
    num_windows = idx2d.shape[0]
    steps_per_worker = num_windows // NUM_WORKERS
    idx3d = idx2d.reshape(NUM_WORKERS, steps_per_worker, WINDOW)

    mesh = plsc.VectorSubcoreMesh(
        core_axis_name="core", subcore_axis_name="subcore"
    )

    @pl.kernel(
        out_type=jax.ShapeDtypeStruct(
            (num_windows, WINDOW, EMBEDDING_DIM), jnp.float32
        ),
        mesh=mesh,
        scratch_types=[
            pltpu.VMEM((steps_per_worker, WINDOW), jnp.int32),
            pltpu.VMEM((NBUF, WINDOW, EMBEDDING_DIM), jnp.float32),
            pltpu.SemaphoreType.DMA((NBUF,)),
            pltpu.SemaphoreType.DMA((NBUF // 2,)),
        ],
    )
    def gather_kernel(table_hbm, idx_hbm, out_hbm, idx_v, bufs, gsem, osem):
        wid = lax.axis_index("subcore") * NUM_CORES + lax.axis_index("core")
        row0 = wid * steps_per_worker

        pltpu.sync_copy(idx_hbm.at[wid], idx_v)

        for b in range(NBUF):
            pltpu.async_copy(table_hbm.at[idx_v.at[b]], bufs.at[b], gsem.at[b])

        def pair_out(p, j):
            for q in range(2):
                pltpu.make_async_copy(
                    table_hbm.at[idx_v.at[j + q]],
                    bufs.at[2 * p + q],
                    gsem.at[2 * p + q],
                ).wait()
            pltpu.async_copy(
                bufs.at[pl.ds(2 * p, 2)],
                out_hbm.at[pl.ds(row0 + j, 2)],
                osem.at[p],
            )

        def pair_out_wait(p, j):
            pltpu.make_async_copy(
                bufs.at[pl.ds(2 * p, 2)],
                out_hbm.at[pl.ds(row0 + j, 2)],
                osem.at[p],
            ).wait()

        @pl.loop(0, steps_per_worker - NBUF, step=NBUF)
        def _(jo):
            for p in range(NBUF // 2):
                j = jo + 2 * p
                pair_out(p, j)
                pair_out_wait(p, j)
                for q in range(2):
                    pltpu.async_copy(
                        table_hbm.at[idx_v.at[j + NBUF + q]],
                        bufs.at[2 * p + q],
                        gsem.at[2 * p + q],
                    )

        jt = steps_per_worker - NBUF
        for p in range(NBUF // 2):
            pair_out(p, jt + 2 * p)
        for p in range(NBUF // 2):
            pair_out_wait(p, jt + 2 * p)

    return gather_kernel(table, idx3d)


def kernel(sentences_indices, embedding_table):
    batch, hist = sentences_indices.shape
    num_indices = batch * hist
    num_windows = num_indices // WINDOW
    idx2d = sentences_indices.reshape(num_windows, WINDOW).astype(jnp.int32)

    devs = jax.devices()
    if len(devs) >= 2:
        dmesh = Mesh(np.array(devs[:2]), ("d",))
        idx_sh = jax.device_put(
            idx2d.reshape(2, num_windows // 2, WINDOW),
            NamedSharding(dmesh, P("d")),
        )
        tab_sh = jax.device_put(embedding_table, NamedSharding(dmesh, P()))

        def local(tab, idxs):
            return _sc_gather(tab, idxs[0])[None]

        out = _shard_map(
            local,
            mesh=dmesh,
            in_specs=(P(), P("d")),
            out_specs=P("d"),
            check_vma=False,
        )(tab_sh, idx_sh)
    else:
        out = _sc_gather(embedding_table, idx2d)

    return out.reshape(batch, hist, EMBEDDING_DIM)
